# baseline (speedup 1.0000x reference)
import functools

import jax
import jax.numpy as jnp
import numpy as _np
from jax import lax
from jax.experimental import pallas as pl
from jax.experimental.pallas import tpu as pltpu
from jax.experimental.pallas import tpu_sc as plsc

VOCAB = 1000000
EMBED = 32
NUM_CLASS = 20
HIDDEN = 128
BATCH = 16384
SEQ = 20
EPS = 1e-5

N_IDX = BATCH * SEQ
NUM_WORKERS = 32
PER_W = N_IDX // NUM_WORKERS
CHUNK = 128
NCHUNK = PER_W // CHUNK

_SC_MESH = plsc.VectorSubcoreMesh(
    core_axis_name="c", subcore_axis_name="s", num_cores=2, num_subcores=16
)


BAGS_W = BATCH // NUM_WORKERS
TJ_W = BAGS_W // CHUNK
NBUF = 8


@functools.partial(
    pl.kernel,
    mesh=_SC_MESH,
    compiler_params=pltpu.CompilerParams(use_tc_tiling_on_sc=False),
    out_type=jax.ShapeDtypeStruct((BATCH // 4, 4, EMBED), jnp.float32),
    scratch_types=[
        pltpu.VMEM((TJ_W * 24, CHUNK), jnp.int32),
        pltpu.VMEM((TJ_W, CHUNK), jnp.int32),
        pltpu.VMEM((CHUNK, EMBED), jnp.float32),
        pltpu.VMEM((CHUNK, EMBED), jnp.float32),
        pltpu.VMEM((CHUNK, EMBED), jnp.float32),
        pltpu.VMEM((CHUNK, EMBED), jnp.float32),
        pltpu.VMEM((CHUNK, EMBED), jnp.float32),
        pltpu.VMEM((CHUNK, EMBED), jnp.float32),
        pltpu.VMEM((CHUNK, EMBED), jnp.float32),
        pltpu.VMEM((CHUNK, EMBED), jnp.float32),
        pltpu.VMEM((CHUNK, EMBED), jnp.float32),
        pltpu.VMEM_SHARED((BATCH // 2, EMBED), jnp.float32),
    ] + [pltpu.SemaphoreType.DMA] * 16,
)
def _sc_pool(table_hbm, xv_hbm, seg_hbm, out_hbm,
             xbuf, seg_v, r0, r1, r2, r3, r4, r5, r6, r7, zbuf, acc,
             g0, g1, g2, g3, g4, g5, g6, g7,
             s0, s1_, s2_, s3, s4, s5, s6, s7):
    wid = lax.axis_index("s") * 2 + lax.axis_index("c")
    sub = lax.axis_index("s")
    rows = (r0, r1, r2, r3, r4, r5, r6, r7)
    gsem = (g0, g1, g2, g3, g4, g5, g6, g7)
    ssem = (s0, s1_, s2_, s3, s4, s5, s6, s7)

    pltpu.sync_copy(seg_hbm.at[wid], seg_v)
    for tj in range(TJ_W):
        for ti in range(3):
            pltpu.sync_copy(
                xv_hbm.at[ti, wid * TJ_W + tj],
                xbuf.at[pl.ds(24 * tj + 8 * ti, 8)],
            )

    zero = jnp.zeros((1, 16), jnp.float32)

    @pl.loop(0, CHUNK)
    def _(r):
        zbuf[pl.ds(r, 1), pl.ds(0, 16)] = zero
        zbuf[pl.ds(r, 1), pl.ds(16, 16)] = zero

    for k in range(TJ_W):
        pltpu.sync_copy(zbuf, acc.at[pl.ds(sub * BAGS_W + k * CHUNK, CHUNK)])

    def xrow(c):
        return 24 * (c // SEQ) + (c % SEQ)

    def gather(c, j):
        pltpu.async_copy(table_hbm.at[xbuf.at[xrow(c)]], rows[j], gsem[j])

    for j in range(NBUF):
        gather(j, j)

    @pl.loop(0, NCHUNK // NBUF)
    def _(p):
        c0 = p * NBUF
        for j in range(NBUF):
            c = c0 + j
            pltpu.make_async_copy(table_hbm.at[xbuf.at[xrow(c)]],
                                  rows[j], gsem[j]).wait()
            pltpu.async_copy(rows[j], acc.at[seg_v.at[c // SEQ]],
                             ssem[j], add=True)
        for j in range(NBUF):
            c = c0 + j
            pltpu.make_async_copy(rows[j], acc.at[seg_v.at[c // SEQ]],
                                  ssem[j]).wait()

            @pl.when(c + NBUF < NCHUNK)
            def _():
                gather(c + NBUF, j)

    pltpu.sync_copy(
        acc.at[pl.ds(sub * BAGS_W, BAGS_W)],
        out_hbm.at[pl.ds(TJ_W * CHUNK * (wid // 4), BAGS_W), wid % 4],
    )


_TCOLS = 65536
_TQ = _TCOLS // 4
_TGRID = (VOCAB + _TCOLS - 1) // _TCOLS
_TROWS = _TQ * _TGRID


def _transpose_body(in_ref, o_ref):
    x = in_ref[...]
    v = jnp.concatenate(
        [x[:, a * _TQ:(a + 1) * _TQ] for a in range(4)], axis=0
    )
    o_ref[...] = v.T


def _tc_transpose(table_t):
    return pl.pallas_call(
        _transpose_body,
        grid=(_TGRID,),
        in_specs=[pl.BlockSpec((EMBED, _TCOLS), lambda i: (0, i))],
        out_specs=pl.BlockSpec((_TQ, 4 * EMBED), lambda i: (i, 0)),
        out_shape=jax.ShapeDtypeStruct((_TROWS, 4 * EMBED), jnp.float32),
        compiler_params=pltpu.CompilerParams(
            dimension_semantics=("parallel",)
        ),
    )(table_t)


def _permute_idx(v):
    return _TCOLS * (v // _TCOLS) + 4 * (v % _TQ) + (v % _TCOLS) // _TQ


_MLP_COLS = 2048
_MLP_BR = _MLP_COLS // 4


def _mlp_body(p_ref, s1_ref, b1_ref, w1_ref, c1_ref, w2_ref, c2_ref, o_ref):
    p4 = p_ref[...]
    v = p4.T
    pt = jnp.concatenate(
        [v[a * EMBED:(a + 1) * EMBED] for a in range(4)], axis=1
    )
    h0 = jnp.maximum(pt * s1_ref[...] + b1_ref[...], 0.0)
    h1 = jnp.dot(w1_ref[...], h0, preferred_element_type=jnp.float32)
    h1 = jnp.maximum(h1 + c1_ref[...], 0.0)
    o_ref[...] = (
        jnp.dot(w2_ref[...], h1, preferred_element_type=jnp.float32) + c2_ref[...]
    )


def _tc_mlp(pooled4, s1, b1, w1, c1, w2, c2):
    grid = (BATCH // _MLP_COLS,)
    full = lambda r, c: pl.BlockSpec((r, c), lambda i: (0, 0))
    return pl.pallas_call(
        _mlp_body,
        grid=grid,
        in_specs=[
            pl.BlockSpec((_MLP_BR, 4 * EMBED), lambda i: (i, 0)),
            full(EMBED, 1),
            full(EMBED, 1),
            full(HIDDEN, EMBED),
            full(HIDDEN, 1),
            full(NUM_CLASS, HIDDEN),
            full(NUM_CLASS, 1),
        ],
        out_specs=pl.BlockSpec((NUM_CLASS, _MLP_COLS), lambda i: (0, i)),
        out_shape=jax.ShapeDtypeStruct((NUM_CLASS, BATCH), jnp.float32),
    )(pooled4, s1, b1, w1, c1, w2, c2)


_SEG = (
    (_np.arange(NUM_WORKERS) // 2 * BAGS_W)[:, None, None]
    + (_np.arange(TJ_W) * CHUNK)[None, :, None]
    + _np.arange(CHUNK)[None, None, :]
).astype(_np.int32)


def kernel(x, table, bn1_w, bn1_b, fc1_w, fc1_b, bn2_w, bn2_b, fc2_w, fc2_b):
    inv = 1.0 / jnp.sqrt(1.0 + EPS)
    s1 = (bn1_w * inv / SEQ).reshape(EMBED, 1)
    s2 = bn2_w * inv
    b1 = bn1_b.reshape(EMBED, 1)
    w1 = fc1_w * s2[:, None]
    c1 = (fc1_b * s2 + bn2_b).reshape(HIDDEN, 1)
    w2 = fc2_w
    c2 = fc2_b.reshape(NUM_CLASS, 1)

    xpt = _permute_idx(x.T)
    xp24 = jnp.concatenate([xpt, xpt[:4]], axis=0)
    xv = xp24.reshape(3, 8, BATCH // CHUNK, CHUNK).transpose(0, 2, 1, 3)

    table_lin = _tc_transpose(table.T).reshape(4 * _TROWS, EMBED)
    pooled4 = _sc_pool(table_lin, xv, jnp.asarray(_SEG))
    out_t = _tc_mlp(pooled4.reshape(BATCH // 4, 4 * EMBED),
                    s1, b1, w1, c1, w2, c2)
    return out_t.T


# speedup vs baseline: 1.1120x; 1.1120x over previous
import functools

import jax
import jax.numpy as jnp
import numpy as _np
from jax import lax
from jax.experimental import pallas as pl
from jax.experimental.pallas import tpu as pltpu
from jax.experimental.pallas import tpu_sc as plsc

VOCAB = 1000000
EMBED = 32
NUM_CLASS = 20
HIDDEN = 128
BATCH = 16384
SEQ = 20
EPS = 1e-5

N_IDX = BATCH * SEQ
NUM_WORKERS = 32
PER_W = N_IDX // NUM_WORKERS
CHUNK = 128
NCHUNK = PER_W // CHUNK

_SC_MESH = plsc.VectorSubcoreMesh(
    core_axis_name="c", subcore_axis_name="s", num_cores=2, num_subcores=16
)


BAGS_W = BATCH // NUM_WORKERS
TJ_W = BAGS_W // CHUNK
NBUF = 8


@functools.partial(
    pl.kernel,
    mesh=_SC_MESH,
    compiler_params=pltpu.CompilerParams(use_tc_tiling_on_sc=False),
    out_type=jax.ShapeDtypeStruct((BATCH // 4, 4 * EMBED), jnp.float32),
    scratch_types=[
        pltpu.VMEM((TJ_W * 24, CHUNK), jnp.int32),
        pltpu.VMEM((TJ_W, CHUNK), jnp.int32),
        pltpu.VMEM((CHUNK, EMBED), jnp.float32),
        pltpu.VMEM((CHUNK, EMBED), jnp.float32),
        pltpu.VMEM((CHUNK, EMBED), jnp.float32),
        pltpu.VMEM((CHUNK, EMBED), jnp.float32),
        pltpu.VMEM((CHUNK, EMBED), jnp.float32),
        pltpu.VMEM((CHUNK, EMBED), jnp.float32),
        pltpu.VMEM((CHUNK, EMBED), jnp.float32),
        pltpu.VMEM((CHUNK, EMBED), jnp.float32),
        pltpu.VMEM((CHUNK, EMBED), jnp.float32),
        pltpu.VMEM_SHARED((BATCH // 2, EMBED), jnp.float32),
    ] + [pltpu.SemaphoreType.DMA] * 16,
)
def _sc_pool(table_hbm, xv_hbm, seg_hbm, out_hbm,
             xbuf, seg_v, r0, r1, r2, r3, r4, r5, r6, r7, zbuf, acc,
             g0, g1, g2, g3, g4, g5, g6, g7,
             s0, s1_, s2_, s3, s4, s5, s6, s7):
    wid = lax.axis_index("s") * 2 + lax.axis_index("c")
    sub = lax.axis_index("s")
    rows = (r0, r1, r2, r3, r4, r5, r6, r7)
    gsem = (g0, g1, g2, g3, g4, g5, g6, g7)
    ssem = (s0, s1_, s2_, s3, s4, s5, s6, s7)

    pltpu.sync_copy(seg_hbm.at[wid], seg_v)
    for tj in range(TJ_W):
        for ti in range(3):
            pltpu.sync_copy(
                xv_hbm.at[ti, wid * TJ_W + tj],
                xbuf.at[pl.ds(24 * tj + 8 * ti, 8)],
            )

    zero = jnp.zeros((1, 16), jnp.float32)

    @pl.loop(0, CHUNK)
    def _(r):
        zbuf[pl.ds(r, 1), pl.ds(0, 16)] = zero
        zbuf[pl.ds(r, 1), pl.ds(16, 16)] = zero

    for k in range(TJ_W):
        pltpu.sync_copy(zbuf, acc.at[pl.ds(sub * BAGS_W + k * CHUNK, CHUNK)])

    def xrow(c):
        return 24 * (c // SEQ) + (c % SEQ)

    def gather(c, j):
        pltpu.async_copy(table_hbm.at[xbuf.at[xrow(c)]], rows[j], gsem[j])

    for j in range(NBUF):
        gather(j, j)

    @pl.loop(0, NCHUNK // NBUF)
    def _(p):
        c0 = p * NBUF
        for j in range(NBUF):
            c = c0 + j
            pltpu.make_async_copy(table_hbm.at[xbuf.at[xrow(c)]],
                                  rows[j], gsem[j]).wait()
            pltpu.async_copy(rows[j], acc.at[seg_v.at[c // SEQ]],
                             ssem[j], add=True)
        for j in range(NBUF):
            c = c0 + j
            pltpu.make_async_copy(rows[j], acc.at[seg_v.at[c // SEQ]],
                                  ssem[j]).wait()

            @pl.when(c + NBUF < NCHUNK)
            def _():
                gather(c + NBUF, j)

    pltpu.sync_copy(
        acc.at[pl.ds(sub * BAGS_W, BAGS_W)],
        out_hbm.at[pl.ds(TJ_W * CHUNK * (wid // 4), BAGS_W),
                   pl.ds(EMBED * (wid % 4), EMBED)],
    )


_TCOLS = 65536
_TQ = _TCOLS // 4
_TGRID = (VOCAB + _TCOLS - 1) // _TCOLS
_TROWS = _TQ * _TGRID


def _transpose_body(in_ref, o_ref):
    x = in_ref[...]
    v = jnp.concatenate(
        [x[:, a * _TQ:(a + 1) * _TQ] for a in range(4)], axis=0
    )
    o_ref[...] = v.T


def _tc_transpose(table_t):
    return pl.pallas_call(
        _transpose_body,
        grid=(_TGRID,),
        in_specs=[pl.BlockSpec((EMBED, _TCOLS), lambda i: (0, i))],
        out_specs=pl.BlockSpec((_TQ, 4 * EMBED), lambda i: (i, 0)),
        out_shape=jax.ShapeDtypeStruct((_TROWS, 4 * EMBED), jnp.float32),
        compiler_params=pltpu.CompilerParams(
            dimension_semantics=("parallel",)
        ),
    )(table_t)


def _permute_idx(v):
    return _TCOLS * (v // _TCOLS) + 4 * (v % _TQ) + (v % _TCOLS) // _TQ


_MLP_COLS = 2048
_MLP_BR = _MLP_COLS // 4


def _mlp_body(p_ref, s1_ref, b1_ref, w1_ref, c1_ref, w2_ref, c2_ref, o_ref):
    p4 = p_ref[...]
    v = p4.T
    pt = jnp.concatenate(
        [v[a * EMBED:(a + 1) * EMBED] for a in range(4)], axis=1
    )
    h0 = jnp.maximum(pt * s1_ref[...] + b1_ref[...], 0.0)
    h1 = jnp.dot(w1_ref[...], h0, preferred_element_type=jnp.float32)
    h1 = jnp.maximum(h1 + c1_ref[...], 0.0)
    o_ref[...] = (
        jnp.dot(w2_ref[...], h1, preferred_element_type=jnp.float32) + c2_ref[...]
    )


def _tc_mlp(pooled4, s1, b1, w1, c1, w2, c2):
    grid = (BATCH // _MLP_COLS,)
    full = lambda r, c: pl.BlockSpec((r, c), lambda i: (0, 0))
    return pl.pallas_call(
        _mlp_body,
        grid=grid,
        in_specs=[
            pl.BlockSpec((_MLP_BR, 4 * EMBED), lambda i: (i, 0)),
            full(EMBED, 1),
            full(EMBED, 1),
            full(HIDDEN, EMBED),
            full(HIDDEN, 1),
            full(NUM_CLASS, HIDDEN),
            full(NUM_CLASS, 1),
        ],
        out_specs=pl.BlockSpec((NUM_CLASS, _MLP_COLS), lambda i: (0, i)),
        out_shape=jax.ShapeDtypeStruct((NUM_CLASS, BATCH), jnp.float32),
    )(pooled4, s1, b1, w1, c1, w2, c2)


_SEG = (
    (_np.arange(NUM_WORKERS) // 2 * BAGS_W)[:, None, None]
    + (_np.arange(TJ_W) * CHUNK)[None, :, None]
    + _np.arange(CHUNK)[None, None, :]
).astype(_np.int32)


def kernel(x, table, bn1_w, bn1_b, fc1_w, fc1_b, bn2_w, bn2_b, fc2_w, fc2_b):
    inv = 1.0 / jnp.sqrt(1.0 + EPS)
    s1 = (bn1_w * inv / SEQ).reshape(EMBED, 1)
    s2 = bn2_w * inv
    b1 = bn1_b.reshape(EMBED, 1)
    w1 = fc1_w * s2[:, None]
    c1 = (fc1_b * s2 + bn2_b).reshape(HIDDEN, 1)
    w2 = fc2_w
    c2 = fc2_b.reshape(NUM_CLASS, 1)

    xpt = _permute_idx(x.T)
    xp24 = jnp.concatenate([xpt, xpt[:4]], axis=0)
    xv = xp24.reshape(3, 8, BATCH // CHUNK, CHUNK).transpose(0, 2, 1, 3)

    table_lin = _tc_transpose(table.T).reshape(4 * _TROWS, EMBED)
    pooled4 = _sc_pool(table_lin, xv, jnp.asarray(_SEG))
    out_t = _tc_mlp(pooled4, s1, b1, w1, c1, w2, c2)
    return out_t.T


# speedup vs baseline: 1.1301x; 1.0163x over previous
import functools

import jax
import jax.numpy as jnp
import numpy as _np
from jax import lax
from jax.experimental import pallas as pl
from jax.experimental.pallas import tpu as pltpu
from jax.experimental.pallas import tpu_sc as plsc

VOCAB = 1000000
EMBED = 32
NUM_CLASS = 20
HIDDEN = 128
BATCH = 16384
SEQ = 20
EPS = 1e-5

N_IDX = BATCH * SEQ
NUM_WORKERS = 32
PER_W = N_IDX // NUM_WORKERS
CHUNK = 128
NCHUNK = PER_W // CHUNK

_SC_MESH = plsc.VectorSubcoreMesh(
    core_axis_name="c", subcore_axis_name="s", num_cores=2, num_subcores=16
)


BAGS_W = BATCH // NUM_WORKERS
TJ_W = BAGS_W // CHUNK
NBUF = 8


@functools.partial(
    pl.kernel,
    mesh=_SC_MESH,
    compiler_params=pltpu.CompilerParams(use_tc_tiling_on_sc=False),
    out_type=jax.ShapeDtypeStruct((BATCH // 4, 4 * EMBED), jnp.float32),
    scratch_types=[
        pltpu.VMEM((TJ_W * 24, CHUNK), jnp.int32),
        pltpu.VMEM((TJ_W, CHUNK), jnp.int32),
        pltpu.VMEM((CHUNK, EMBED), jnp.float32),
        pltpu.VMEM((CHUNK, EMBED), jnp.float32),
        pltpu.VMEM((CHUNK, EMBED), jnp.float32),
        pltpu.VMEM((CHUNK, EMBED), jnp.float32),
        pltpu.VMEM((CHUNK, EMBED), jnp.float32),
        pltpu.VMEM((CHUNK, EMBED), jnp.float32),
        pltpu.VMEM((CHUNK, EMBED), jnp.float32),
        pltpu.VMEM((CHUNK, EMBED), jnp.float32),
        pltpu.VMEM((CHUNK, EMBED), jnp.float32),
        pltpu.VMEM_SHARED((BATCH // 2, EMBED), jnp.float32),
    ] + [pltpu.SemaphoreType.DMA] * 16,
)
def _sc_pool(table_hbm, xv_hbm, seg_hbm, out_hbm,
             xbuf, seg_v, r0, r1, r2, r3, r4, r5, r6, r7, zbuf, acc,
             g0, g1, g2, g3, g4, g5, g6, g7,
             s0, s1_, s2_, s3, s4, s5, s6, s7):
    wid = lax.axis_index("s") * 2 + lax.axis_index("c")
    sub = lax.axis_index("s")
    rows = (r0, r1, r2, r3, r4, r5, r6, r7)
    gsem = (g0, g1, g2, g3, g4, g5, g6, g7)
    ssem = (s0, s1_, s2_, s3, s4, s5, s6, s7)

    pltpu.sync_copy(seg_hbm.at[wid], seg_v)
    for tj in range(TJ_W):
        for ti in range(3):
            pltpu.sync_copy(
                xv_hbm.at[ti, wid * TJ_W + tj],
                xbuf.at[pl.ds(24 * tj + 8 * ti, 8)],
            )

    zero = jnp.zeros((1, 16), jnp.float32)

    @pl.loop(0, CHUNK)
    def _(r):
        zbuf[pl.ds(r, 1), pl.ds(0, 16)] = zero
        zbuf[pl.ds(r, 1), pl.ds(16, 16)] = zero

    for k in range(TJ_W):
        pltpu.sync_copy(zbuf, acc.at[pl.ds(sub * BAGS_W + k * CHUNK, CHUNK)])

    def xrow(c):
        return 24 * (c // SEQ) + (c % SEQ)

    def gather(c, j):
        pltpu.async_copy(table_hbm.at[xbuf.at[xrow(c)]], rows[j], gsem[j])

    for j in range(NBUF):
        gather(j, j)

    @pl.loop(0, NCHUNK // NBUF)
    def _(p):
        c0 = p * NBUF
        for j in range(NBUF):
            c = c0 + j
            pltpu.make_async_copy(table_hbm.at[xbuf.at[xrow(c)]],
                                  rows[j], gsem[j]).wait()
            pltpu.async_copy(rows[j], acc.at[seg_v.at[c // SEQ]],
                             ssem[j], add=True)
        for j in range(NBUF):
            c = c0 + j
            pltpu.make_async_copy(rows[j], acc.at[seg_v.at[c // SEQ]],
                                  ssem[j]).wait()

            @pl.when(c + NBUF < NCHUNK)
            def _():
                gather(c + NBUF, j)

    pltpu.sync_copy(
        acc.at[pl.ds(sub * BAGS_W, BAGS_W)],
        out_hbm.at[pl.ds(TJ_W * CHUNK * (wid // 4), BAGS_W),
                   pl.ds(EMBED * (wid % 4), EMBED)],
    )


_TCOLS = 65536
_TQ = _TCOLS // 4
_TGRID = (VOCAB + _TCOLS - 1) // _TCOLS
_TROWS = _TQ * _TGRID


def _transpose_body(in_ref, o_ref):
    x = in_ref[...]
    v = jnp.concatenate(
        [x[:, a * _TQ:(a + 1) * _TQ] for a in range(4)], axis=0
    )
    o_ref[...] = v.T


def _tc_transpose(table_t):
    return pl.pallas_call(
        _transpose_body,
        grid=(_TGRID,),
        in_specs=[pl.BlockSpec((EMBED, _TCOLS), lambda i: (0, i))],
        out_specs=pl.BlockSpec((_TQ, 4 * EMBED), lambda i: (i, 0)),
        out_shape=jax.ShapeDtypeStruct((_TROWS, 4 * EMBED), jnp.float32),
        compiler_params=pltpu.CompilerParams(
            dimension_semantics=("parallel",)
        ),
    )(table_t)


def _permute_idx(v):
    return _TCOLS * (v // _TCOLS) + 4 * (v % _TQ) + (v % _TCOLS) // _TQ


_MLP_COLS = 4096
_MLP_BR = _MLP_COLS // 4


def _mlp_body(p_ref, s1_ref, b1_ref, w1_ref, c1_ref, w2_ref, c2_ref, o_ref):
    p4 = p_ref[...]
    v = p4.T
    pt = jnp.concatenate(
        [v[a * EMBED:(a + 1) * EMBED] for a in range(4)], axis=1
    )
    h0 = jnp.maximum(pt * s1_ref[...] + b1_ref[...], 0.0)
    h1 = jnp.dot(w1_ref[...], h0, preferred_element_type=jnp.float32)
    h1 = jnp.maximum(h1 + c1_ref[...], 0.0)
    o_ref[...] = (
        jnp.dot(w2_ref[...], h1, preferred_element_type=jnp.float32) + c2_ref[...]
    )


def _tc_mlp(pooled4, s1, b1, w1, c1, w2, c2):
    grid = (BATCH // _MLP_COLS,)
    full = lambda r, c: pl.BlockSpec((r, c), lambda i: (0, 0))
    return pl.pallas_call(
        _mlp_body,
        grid=grid,
        in_specs=[
            pl.BlockSpec((_MLP_BR, 4 * EMBED), lambda i: (i, 0)),
            full(EMBED, 1),
            full(EMBED, 1),
            full(HIDDEN, EMBED),
            full(HIDDEN, 1),
            full(NUM_CLASS, HIDDEN),
            full(NUM_CLASS, 1),
        ],
        out_specs=pl.BlockSpec((NUM_CLASS, _MLP_COLS), lambda i: (0, i)),
        out_shape=jax.ShapeDtypeStruct((NUM_CLASS, BATCH), jnp.float32),
    )(pooled4, s1, b1, w1, c1, w2, c2)


_SEG = (
    (_np.arange(NUM_WORKERS) // 2 * BAGS_W)[:, None, None]
    + (_np.arange(TJ_W) * CHUNK)[None, :, None]
    + _np.arange(CHUNK)[None, None, :]
).astype(_np.int32)


def kernel(x, table, bn1_w, bn1_b, fc1_w, fc1_b, bn2_w, bn2_b, fc2_w, fc2_b):
    inv = 1.0 / jnp.sqrt(1.0 + EPS)
    s1 = (bn1_w * inv / SEQ).reshape(EMBED, 1)
    s2 = bn2_w * inv
    b1 = bn1_b.reshape(EMBED, 1)
    w1 = fc1_w * s2[:, None]
    c1 = (fc1_b * s2 + bn2_b).reshape(HIDDEN, 1)
    w2 = fc2_w
    c2 = fc2_b.reshape(NUM_CLASS, 1)

    xpt = _permute_idx(x.T)
    xp24 = jnp.concatenate([xpt, xpt[:4]], axis=0)
    xv = xp24.reshape(3, 8, BATCH // CHUNK, CHUNK).transpose(0, 2, 1, 3)

    table_lin = _tc_transpose(table.T).reshape(4 * _TROWS, EMBED)
    pooled4 = _sc_pool(table_lin, xv, jnp.asarray(_SEG))
    out_t = _tc_mlp(pooled4, s1, b1, w1, c1, w2, c2)
    return out_t.T
